# Initial kernel scaffold; baseline (speedup 1.0000x reference)
import jax
import jax.numpy as jnp
from jax.experimental import pallas as pl


def kernel(features, p, batch_ids):
    raise NotImplementedError("write your pallas kernel here")


# speedup vs baseline: 10.8099x; 10.8099x over previous
import jax
import jax.numpy as jnp
from jax.experimental import pallas as pl
from jax.experimental.pallas import tpu as pltpu

N = 32768
D = 256
B = 16
EPS = 1e-06
BLK = 2048
G = N // BLK


def _body(p_ref, f_ref, ids_ref, out_ref, acc_ref, cnt_ref):
    i = pl.program_id(0)

    @pl.when(i == 0)
    def _init():
        acc_ref[...] = jnp.zeros_like(acc_ref)
        cnt_ref[...] = jnp.zeros_like(cnt_ref)

    p = p_ref[0]
    x = jnp.maximum(f_ref[...], EPS)
    pw = jnp.exp(p * jnp.log(x))
    ids = ids_ref[0, 0, :]
    oh = (ids[:, None] == jax.lax.broadcasted_iota(jnp.int32, (BLK, B), 1)
          ).astype(jnp.float32)
    acc_ref[...] += jax.lax.dot_general(
        oh, pw, (((0,), (0,)), ((), ())), preferred_element_type=jnp.float32)
    cnt_ref[...] += jax.lax.dot_general(
        oh, jnp.ones((BLK, 8), jnp.float32), (((0,), (0,)), ((), ())),
        preferred_element_type=jnp.float32)

    @pl.when(i == pl.num_programs(0) - 1)
    def _fin():
        counts = jnp.maximum(cnt_ref[...][:, 0:1], 1.0)
        avg = acc_ref[...] / counts
        out_ref[...] = jnp.exp(jnp.log(avg) / p)


def kernel(features, p, batch_ids):
    ids3 = batch_ids.astype(jnp.int32).reshape(G, 1, BLK)
    return pl.pallas_call(
        _body,
        grid=(G,),
        in_specs=[
            pl.BlockSpec(memory_space=pltpu.SMEM),
            pl.BlockSpec((BLK, D), lambda i: (i, 0)),
            pl.BlockSpec((1, 1, BLK), lambda i: (i, 0, 0)),
        ],
        out_specs=pl.BlockSpec((B, D), lambda i: (0, 0)),
        out_shape=jax.ShapeDtypeStruct((B, D), jnp.float32),
        scratch_shapes=[
            pltpu.VMEM((B, D), jnp.float32),
            pltpu.VMEM((B, 8), jnp.float32),
        ],
    )(p, features, ids3)
